# baseline (speedup 1.0000x reference)
import functools

import jax
import jax.numpy as jnp
from jax import lax
from jax.experimental import pallas as pl
from jax.experimental.pallas import tpu as pltpu
from jax.experimental.pallas import tpu_sc as plsc

_B = 8192
_DIM = 256
_N_EMBED = 8192
_DECAY = 0.99
_ROW_BLK = 256
_N_BLK = _B // _ROW_BLK


_TILE = 2048
_N_TILE = _N_EMBED // _TILE


def _argmin_body(x_ref, x2_ref, e_ref, e2_ref, cs_ref,
                 ind_ref, nsmall_ref, loss_ref, counts_acc, loss_acc):
    i = pl.program_id(0)
    x = x_ref[...]
    x2 = x2_ref[...]

    rv = jnp.full((_ROW_BLK, 1), jnp.inf, jnp.float32)
    rl = jnp.zeros((_ROW_BLK, 1), jnp.float32)
    ri = jnp.zeros((_ROW_BLK,), jnp.int32)
    for t in range(_N_TILE):
        e_t = e_ref[:, t * _TILE:(t + 1) * _TILE]
        e2_t = e2_ref[:, t * _TILE:(t + 1) * _TILE]
        mm = jnp.dot(x, e_t, preferred_element_type=jnp.float32)
        dist = (x2 - 2.0 * mm) + e2_t
        v = jnp.min(dist, axis=1, keepdims=True)
        iota = lax.broadcasted_iota(jnp.int32, dist.shape, 1)
        idx = jnp.min(jnp.where(dist == v, iota, _TILE), axis=1) + t * _TILE
        upd = v < rv
        rv = jnp.where(upd, v.astype(jnp.bfloat16).astype(jnp.float32), rv)
        rl = jnp.where(upd, v, rl)
        ri = jnp.where(upd[:, 0], idx, ri)
    ind = ri
    minval = rl
    ind_ref[0, 0, :] = ind

    full_iota = lax.broadcasted_iota(jnp.int32, (_ROW_BLK, _N_EMBED), 1)
    onehot = (full_iota == ind[:, None]).astype(jnp.float32)
    part = jnp.sum(onehot, axis=0, keepdims=True)

    @pl.when(i == 0)
    def _():
        counts_acc[...] = jnp.zeros_like(counts_acc)
        loss_acc[0] = 0.0

    counts_acc[...] += part
    loss_acc[0] += jnp.sum(minval)

    @pl.when(i == _N_BLK - 1)
    def _():
        counts = counts_acc[...]
        csn = cs_ref[...] * _DECAY + counts * (1.0 - _DECAY)
        nsmall_ref[0] = jnp.sum((csn < 1.0).astype(jnp.float32))
        loss_ref[0] = loss_acc[0] / float(_B * _DIM)


_argmin_call = pl.pallas_call(
    _argmin_body,
    grid=(_N_BLK,),
    in_specs=[
        pl.BlockSpec((_ROW_BLK, _DIM), lambda i: (i, 0)),
        pl.BlockSpec((_ROW_BLK, 1), lambda i: (i, 0)),
        pl.BlockSpec((_DIM, _N_EMBED), lambda i: (0, 0)),
        pl.BlockSpec((1, _N_EMBED), lambda i: (0, 0)),
        pl.BlockSpec((1, _N_EMBED), lambda i: (0, 0)),
    ],
    out_specs=[
        pl.BlockSpec((1, 1, _ROW_BLK), lambda i: (i, 0, 0)),
        pl.BlockSpec(memory_space=pltpu.SMEM),
        pl.BlockSpec(memory_space=pltpu.SMEM),
    ],
    out_shape=[
        jax.ShapeDtypeStruct((_N_BLK, 1, _ROW_BLK), jnp.int32),
        jax.ShapeDtypeStruct((1,), jnp.float32),
        jax.ShapeDtypeStruct((1,), jnp.float32),
    ],
    scratch_shapes=[
        pltpu.VMEM((1, _N_EMBED), jnp.float32),
        pltpu.SMEM((1,), jnp.float32),
    ],
)


@functools.cache
def _sc_gather_fn():
    info = plsc.get_sparse_core_info()
    nc = info.num_cores
    nw = nc * info.num_subcores
    bpw = _B // nw

    @functools.partial(
        pl.kernel,
        mesh=plsc.VectorSubcoreMesh(core_axis_name="c", subcore_axis_name="s"),
        out_type=jax.ShapeDtypeStruct((_B, _DIM), jnp.float32),
        scratch_types=[
            pltpu.VMEM((bpw,), jnp.int32),
            pltpu.VMEM((bpw, _DIM), jnp.float32),
            pltpu.SemaphoreType.DMA,
        ],
    )
    def _sc_gather(table_hbm, idx_hbm, out_hbm, idx_v, rows_v, sem):
        wid = lax.axis_index("s") * nc + lax.axis_index("c")
        base = wid * bpw
        pltpu.sync_copy(idx_hbm.at[pl.ds(base, bpw)], idx_v)
        pltpu.async_copy(table_hbm.at[idx_v], rows_v, sem).wait()
        pltpu.sync_copy(rows_v, out_hbm.at[pl.ds(base, bpw)])

    return _sc_gather


def _st_body(x_ref, q_ref, out_ref):
    x = x_ref[...]
    q = q_ref[...]
    out_ref[...] = x + (q - x)


_st_call = pl.pallas_call(
    _st_body,
    grid=(16,),
    in_specs=[
        pl.BlockSpec((_B // 16, _DIM), lambda i: (i, 0)),
        pl.BlockSpec((_B // 16, _DIM), lambda i: (i, 0)),
    ],
    out_specs=pl.BlockSpec((_B // 16, _DIM), lambda i: (i, 0)),
    out_shape=jax.ShapeDtypeStruct((_B, _DIM), jnp.float32),
)


def kernel(x, embed, cluster_size):
    cs2 = cluster_size.reshape(1, _N_EMBED)
    x2 = jnp.sum(x ** 2, axis=1, keepdims=True)
    e2 = jnp.sum(embed ** 2, axis=0, keepdims=True)
    ind3, nsmall, loss = _argmin_call(x, x2, embed, e2, cs2)
    ind = ind3.reshape(_B)
    table = embed.T
    quantize = _sc_gather_fn()(table, ind)
    quantized_x = _st_call(x, quantize)
    output = ind.reshape(_B, 1).astype(jnp.int64)
    return (output, quantized_x, nsmall.reshape(()), loss.reshape(()))


# speedup vs baseline: 1.1538x; 1.1538x over previous
import functools

import jax
import jax.numpy as jnp
from jax import lax
from jax.experimental import pallas as pl
from jax.experimental.pallas import tpu as pltpu
from jax.experimental.pallas import tpu_sc as plsc

_B = 8192
_DIM = 256
_N_EMBED = 8192
_DECAY = 0.99
_ROW_BLK = 256
_N_BLK = _B // _ROW_BLK
_TILE = 2048
_N_TILE = _N_EMBED // _TILE


def _argmin_body(x_ref, x2_ref, e_ref, e2_ref, ind_ref, cnt_ref, mind_ref):
    x = x_ref[...]
    x2 = x2_ref[...]

    rv = jnp.full((_ROW_BLK, 1), jnp.inf, jnp.float32)
    rl = jnp.zeros((_ROW_BLK, 1), jnp.float32)
    ri = jnp.zeros((_ROW_BLK,), jnp.int32)
    for t in range(_N_TILE):
        e_t = e_ref[:, t * _TILE:(t + 1) * _TILE]
        e2_t = e2_ref[:, t * _TILE:(t + 1) * _TILE]
        mm = jnp.dot(x, e_t, preferred_element_type=jnp.float32)
        dist = (x2 - 2.0 * mm) + e2_t
        v = jnp.min(dist, axis=1, keepdims=True)
        iota = lax.broadcasted_iota(jnp.int32, dist.shape, 1)
        idx = jnp.min(jnp.where(dist == v, iota, _TILE), axis=1) + t * _TILE
        upd = v < rv
        rv = jnp.where(upd, v.astype(jnp.bfloat16).astype(jnp.float32), rv)
        rl = jnp.where(upd, v, rl)
        ri = jnp.where(upd[:, 0], idx, ri)
    ind_ref[0, 0, :] = ri
    mind_ref[0, :, :] = rl

    full_iota = lax.broadcasted_iota(jnp.int32, (_ROW_BLK, _N_EMBED), 1)
    onehot = (full_iota == ri[:, None]).astype(jnp.float32)
    cnt_ref[0, :, :] = jnp.sum(onehot, axis=0, keepdims=True)


_argmin_call = pl.pallas_call(
    _argmin_body,
    grid=(_N_BLK,),
    in_specs=[
        pl.BlockSpec((_ROW_BLK, _DIM), lambda i: (i, 0)),
        pl.BlockSpec((_ROW_BLK, 1), lambda i: (i, 0)),
        pl.BlockSpec((_DIM, _N_EMBED), lambda i: (0, 0)),
        pl.BlockSpec((1, _N_EMBED), lambda i: (0, 0)),
    ],
    out_specs=[
        pl.BlockSpec((1, 1, _ROW_BLK), lambda i: (i, 0, 0)),
        pl.BlockSpec((1, 1, _N_EMBED), lambda i: (i, 0, 0)),
        pl.BlockSpec((1, _ROW_BLK, 1), lambda i: (i, 0, 0)),
    ],
    out_shape=[
        jax.ShapeDtypeStruct((_N_BLK, 1, _ROW_BLK), jnp.int32),
        jax.ShapeDtypeStruct((_N_BLK, 1, _N_EMBED), jnp.float32),
        jax.ShapeDtypeStruct((_N_BLK, _ROW_BLK, 1), jnp.float32),
    ],
    compiler_params=pltpu.CompilerParams(
        dimension_semantics=("parallel",),
    ),
)


def _stats_body(cnt_ref, mind_ref, cs_ref, nsmall_ref, loss_ref):
    counts = jnp.sum(cnt_ref[...], axis=0, keepdims=True)
    csn = cs_ref[...] * _DECAY + counts * (1.0 - _DECAY)
    nsmall_ref[0] = jnp.sum((csn < 1.0).astype(jnp.float32))
    loss_ref[0] = jnp.sum(mind_ref[...]) / float(_B * _DIM)


_stats_call = pl.pallas_call(
    _stats_body,
    in_specs=[
        pl.BlockSpec((_N_BLK, _N_EMBED), lambda: (0, 0)),
        pl.BlockSpec((_B, 1), lambda: (0, 0)),
        pl.BlockSpec((1, _N_EMBED), lambda: (0, 0)),
    ],
    out_specs=[
        pl.BlockSpec(memory_space=pltpu.SMEM),
        pl.BlockSpec(memory_space=pltpu.SMEM),
    ],
    out_shape=[
        jax.ShapeDtypeStruct((1,), jnp.float32),
        jax.ShapeDtypeStruct((1,), jnp.float32),
    ],
)


@functools.cache
def _sc_gather_fn():
    info = plsc.get_sparse_core_info()
    nc = info.num_cores
    nw = nc * info.num_subcores
    bpw = _B // nw

    @functools.partial(
        pl.kernel,
        mesh=plsc.VectorSubcoreMesh(core_axis_name="c", subcore_axis_name="s"),
        out_type=jax.ShapeDtypeStruct((_B, _DIM), jnp.float32),
        scratch_types=[
            pltpu.VMEM((bpw,), jnp.int32),
            pltpu.VMEM((bpw, _DIM), jnp.float32),
            pltpu.SemaphoreType.DMA,
        ],
    )
    def _sc_gather(table_hbm, idx_hbm, out_hbm, idx_v, rows_v, sem):
        wid = lax.axis_index("s") * nc + lax.axis_index("c")
        base = wid * bpw
        pltpu.sync_copy(idx_hbm.at[pl.ds(base, bpw)], idx_v)
        pltpu.async_copy(table_hbm.at[idx_v], rows_v, sem).wait()
        pltpu.sync_copy(rows_v, out_hbm.at[pl.ds(base, bpw)])

    return _sc_gather


def kernel(x, embed, cluster_size):
    cs2 = cluster_size.reshape(1, _N_EMBED)
    x2 = jnp.sum(x ** 2, axis=1, keepdims=True)
    e2 = jnp.sum(embed ** 2, axis=0, keepdims=True)
    ind3, cnt, mind = _argmin_call(x, x2, embed, e2)
    nsmall, loss = _stats_call(cnt.reshape(_N_BLK, _N_EMBED),
                               mind.reshape(_B, 1), cs2)
    ind = ind3.reshape(_B)
    table = embed.T
    quantized_x = _sc_gather_fn()(table, ind)
    output = ind.reshape(_B, 1).astype(jnp.int64)
    return (output, quantized_x, nsmall.reshape(()), loss.reshape(()))


# speedup vs baseline: 1.2244x; 1.0612x over previous
import functools

import jax
import jax.numpy as jnp
from jax import lax
from jax.experimental import pallas as pl
from jax.experimental.pallas import tpu as pltpu
from jax.experimental.pallas import tpu_sc as plsc

_B = 8192
_DIM = 256
_N_EMBED = 8192
_DECAY = 0.99
_ROW_BLK = 512
_N_BLK = _B // _ROW_BLK
_TILE = 2048
_N_TILE = _N_EMBED // _TILE


def _argmin_body(x_ref, x2_ref, e_ref, e2_ref, ind_ref, cnt_ref, mind_ref):
    x = x_ref[...]
    x2 = x2_ref[...]

    rv = jnp.full((_ROW_BLK, 1), jnp.inf, jnp.float32)
    rl = jnp.zeros((_ROW_BLK, 1), jnp.float32)
    ri = jnp.zeros((_ROW_BLK,), jnp.int32)
    for t in range(_N_TILE):
        e_t = e_ref[:, t * _TILE:(t + 1) * _TILE]
        e2_t = e2_ref[:, t * _TILE:(t + 1) * _TILE]
        mm = jnp.dot(x, e_t, preferred_element_type=jnp.float32)
        dist = (x2 - 2.0 * mm) + e2_t
        v = jnp.min(dist, axis=1, keepdims=True)
        iota = lax.broadcasted_iota(jnp.int32, dist.shape, 1)
        idx = jnp.min(jnp.where(dist == v, iota, _TILE), axis=1) + t * _TILE
        upd = v < rv
        rv = jnp.where(upd, v.astype(jnp.bfloat16).astype(jnp.float32), rv)
        rl = jnp.where(upd, v, rl)
        ri = jnp.where(upd[:, 0], idx, ri)
    ind_ref[0, 0, :] = ri
    mind_ref[0, :, :] = rl

    full_iota = lax.broadcasted_iota(jnp.int32, (_ROW_BLK, _N_EMBED), 1)
    onehot = (full_iota == ri[:, None]).astype(jnp.float32)
    cnt_ref[0, :, :] = jnp.sum(onehot, axis=0, keepdims=True)


_argmin_call = pl.pallas_call(
    _argmin_body,
    grid=(_N_BLK,),
    in_specs=[
        pl.BlockSpec((_ROW_BLK, _DIM), lambda i: (i, 0)),
        pl.BlockSpec((_ROW_BLK, 1), lambda i: (i, 0)),
        pl.BlockSpec((_DIM, _N_EMBED), lambda i: (0, 0)),
        pl.BlockSpec((1, _N_EMBED), lambda i: (0, 0)),
    ],
    out_specs=[
        pl.BlockSpec((1, 1, _ROW_BLK), lambda i: (i, 0, 0)),
        pl.BlockSpec((1, 1, _N_EMBED), lambda i: (i, 0, 0)),
        pl.BlockSpec((1, _ROW_BLK, 1), lambda i: (i, 0, 0)),
    ],
    out_shape=[
        jax.ShapeDtypeStruct((_N_BLK, 1, _ROW_BLK), jnp.int32),
        jax.ShapeDtypeStruct((_N_BLK, 1, _N_EMBED), jnp.float32),
        jax.ShapeDtypeStruct((_N_BLK, _ROW_BLK, 1), jnp.float32),
    ],
    compiler_params=pltpu.CompilerParams(
        dimension_semantics=("parallel",),
    ),
)


def _stats_body(cnt_ref, mind_ref, cs_ref, nsmall_ref, loss_ref):
    counts = jnp.sum(cnt_ref[...], axis=0, keepdims=True)
    csn = cs_ref[...] * _DECAY + counts * (1.0 - _DECAY)
    nsmall_ref[0] = jnp.sum((csn < 1.0).astype(jnp.float32))
    loss_ref[0] = jnp.sum(mind_ref[...]) / float(_B * _DIM)


_stats_call = pl.pallas_call(
    _stats_body,
    in_specs=[
        pl.BlockSpec((_N_BLK, _N_EMBED), lambda: (0, 0)),
        pl.BlockSpec((_B, 1), lambda: (0, 0)),
        pl.BlockSpec((1, _N_EMBED), lambda: (0, 0)),
    ],
    out_specs=[
        pl.BlockSpec(memory_space=pltpu.SMEM),
        pl.BlockSpec(memory_space=pltpu.SMEM),
    ],
    out_shape=[
        jax.ShapeDtypeStruct((1,), jnp.float32),
        jax.ShapeDtypeStruct((1,), jnp.float32),
    ],
)


@functools.cache
def _sc_gather_fn():
    info = plsc.get_sparse_core_info()
    nc = info.num_cores
    nw = nc * info.num_subcores
    bpw = _B // nw

    @functools.partial(
        pl.kernel,
        mesh=plsc.VectorSubcoreMesh(core_axis_name="c", subcore_axis_name="s"),
        out_type=jax.ShapeDtypeStruct((_B, _DIM), jnp.float32),
        scratch_types=[
            pltpu.VMEM((bpw,), jnp.int32),
            pltpu.VMEM((bpw, _DIM), jnp.float32),
            pltpu.SemaphoreType.DMA,
        ],
    )
    def _sc_gather(table_hbm, idx_hbm, out_hbm, idx_v, rows_v, sem):
        wid = lax.axis_index("s") * nc + lax.axis_index("c")
        base = wid * bpw
        pltpu.sync_copy(idx_hbm.at[pl.ds(base, bpw)], idx_v)
        pltpu.async_copy(table_hbm.at[idx_v], rows_v, sem).wait()
        pltpu.sync_copy(rows_v, out_hbm.at[pl.ds(base, bpw)])

    return _sc_gather


def kernel(x, embed, cluster_size):
    cs2 = cluster_size.reshape(1, _N_EMBED)
    x2 = jnp.sum(x ** 2, axis=1, keepdims=True)
    e2 = jnp.sum(embed ** 2, axis=0, keepdims=True)
    ind3, cnt, mind = _argmin_call(x, x2, embed, e2)
    nsmall, loss = _stats_call(cnt.reshape(_N_BLK, _N_EMBED),
                               mind.reshape(_B, 1), cs2)
    ind = ind3.reshape(_B)
    table = embed.T
    quantized_x = _sc_gather_fn()(table, ind)
    output = ind.reshape(_B, 1).astype(jnp.int64)
    return (output, quantized_x, nsmall.reshape(()), loss.reshape(()))


# speedup vs baseline: 1.2799x; 1.0453x over previous
import functools

import jax
import jax.numpy as jnp
from jax import lax
from jax.experimental import pallas as pl
from jax.experimental.pallas import tpu as pltpu
from jax.experimental.pallas import tpu_sc as plsc

_B = 8192
_DIM = 256
_N_EMBED = 8192
_DECAY = 0.99
_ROW_BLK = 1024
_N_BLK = _B // _ROW_BLK
_TILE = 2048
_N_TILE = _N_EMBED // _TILE


def _argmin_body(x_ref, x2_ref, e_ref, e2_ref, ind_ref, cnt_ref, mind_ref):
    x = x_ref[...]
    x2 = x2_ref[...]

    rv = jnp.full((_ROW_BLK, 1), jnp.inf, jnp.float32)
    rl = jnp.zeros((_ROW_BLK, 1), jnp.float32)
    ri = jnp.zeros((_ROW_BLK,), jnp.int32)
    for t in range(_N_TILE):
        e_t = e_ref[:, t * _TILE:(t + 1) * _TILE]
        e2_t = e2_ref[:, t * _TILE:(t + 1) * _TILE]
        mm = jnp.dot(x, e_t, preferred_element_type=jnp.float32)
        dist = (x2 - 2.0 * mm) + e2_t
        v = jnp.min(dist, axis=1, keepdims=True)
        iota = lax.broadcasted_iota(jnp.int32, dist.shape, 1)
        idx = jnp.min(jnp.where(dist == v, iota, _TILE), axis=1) + t * _TILE
        upd = v < rv
        rv = jnp.where(upd, v.astype(jnp.bfloat16).astype(jnp.float32), rv)
        rl = jnp.where(upd, v, rl)
        ri = jnp.where(upd[:, 0], idx, ri)
    ind_ref[0, 0, :] = ri
    mind_ref[0, :, :] = rl

    full_iota = lax.broadcasted_iota(jnp.int32, (_ROW_BLK, _N_EMBED), 1)
    onehot = (full_iota == ri[:, None]).astype(jnp.float32)
    cnt_ref[0, :, :] = jnp.sum(onehot, axis=0, keepdims=True)


_argmin_call = pl.pallas_call(
    _argmin_body,
    grid=(_N_BLK,),
    in_specs=[
        pl.BlockSpec((_ROW_BLK, _DIM), lambda i: (i, 0)),
        pl.BlockSpec((_ROW_BLK, 1), lambda i: (i, 0)),
        pl.BlockSpec((_DIM, _N_EMBED), lambda i: (0, 0)),
        pl.BlockSpec((1, _N_EMBED), lambda i: (0, 0)),
    ],
    out_specs=[
        pl.BlockSpec((1, 1, _ROW_BLK), lambda i: (i, 0, 0)),
        pl.BlockSpec((1, 1, _N_EMBED), lambda i: (i, 0, 0)),
        pl.BlockSpec((1, _ROW_BLK, 1), lambda i: (i, 0, 0)),
    ],
    out_shape=[
        jax.ShapeDtypeStruct((_N_BLK, 1, _ROW_BLK), jnp.int32),
        jax.ShapeDtypeStruct((_N_BLK, 1, _N_EMBED), jnp.float32),
        jax.ShapeDtypeStruct((_N_BLK, _ROW_BLK, 1), jnp.float32),
    ],
    compiler_params=pltpu.CompilerParams(
        dimension_semantics=("parallel",),
    ),
)


def _stats_body(cnt_ref, mind_ref, cs_ref, nsmall_ref, loss_ref):
    counts = jnp.sum(cnt_ref[...], axis=0, keepdims=True)
    csn = cs_ref[...] * _DECAY + counts * (1.0 - _DECAY)
    nsmall_ref[0] = jnp.sum((csn < 1.0).astype(jnp.float32))
    loss_ref[0] = jnp.sum(mind_ref[...]) / float(_B * _DIM)


_stats_call = pl.pallas_call(
    _stats_body,
    in_specs=[
        pl.BlockSpec((_N_BLK, _N_EMBED), lambda: (0, 0)),
        pl.BlockSpec((_B, 1), lambda: (0, 0)),
        pl.BlockSpec((1, _N_EMBED), lambda: (0, 0)),
    ],
    out_specs=[
        pl.BlockSpec(memory_space=pltpu.SMEM),
        pl.BlockSpec(memory_space=pltpu.SMEM),
    ],
    out_shape=[
        jax.ShapeDtypeStruct((1,), jnp.float32),
        jax.ShapeDtypeStruct((1,), jnp.float32),
    ],
)


@functools.cache
def _sc_gather_fn():
    info = plsc.get_sparse_core_info()
    nc = info.num_cores
    nw = nc * info.num_subcores
    bpw = _B // nw

    @functools.partial(
        pl.kernel,
        mesh=plsc.VectorSubcoreMesh(core_axis_name="c", subcore_axis_name="s"),
        out_type=jax.ShapeDtypeStruct((_B, _DIM), jnp.float32),
        scratch_types=[
            pltpu.VMEM((bpw,), jnp.int32),
            pltpu.VMEM((bpw, _DIM), jnp.float32),
            pltpu.SemaphoreType.DMA,
        ],
    )
    def _sc_gather(table_hbm, idx_hbm, out_hbm, idx_v, rows_v, sem):
        wid = lax.axis_index("s") * nc + lax.axis_index("c")
        base = wid * bpw
        pltpu.sync_copy(idx_hbm.at[pl.ds(base, bpw)], idx_v)
        pltpu.async_copy(table_hbm.at[idx_v], rows_v, sem).wait()
        pltpu.sync_copy(rows_v, out_hbm.at[pl.ds(base, bpw)])

    return _sc_gather


def kernel(x, embed, cluster_size):
    cs2 = cluster_size.reshape(1, _N_EMBED)
    x2 = jnp.sum(x ** 2, axis=1, keepdims=True)
    e2 = jnp.sum(embed ** 2, axis=0, keepdims=True)
    ind3, cnt, mind = _argmin_call(x, x2, embed, e2)
    nsmall, loss = _stats_call(cnt.reshape(_N_BLK, _N_EMBED),
                               mind.reshape(_B, 1), cs2)
    ind = ind3.reshape(_B)
    table = embed.T
    quantized_x = _sc_gather_fn()(table, ind)
    output = ind.reshape(_B, 1).astype(jnp.int64)
    return (output, quantized_x, nsmall.reshape(()), loss.reshape(()))
